# baseline (speedup 1.0000x reference)
import functools

import jax
import jax.numpy as jnp
from jax import lax
from jax.experimental import pallas as pl
from jax.experimental.pallas import tpu as pltpu
from jax.experimental.pallas import tpu_sc as plsc

D = 1024
L = 16
NC = 2
NS = 16
NW = NC * NS
SCALE = 32.0

CHUNK = 16
RING_G = 4
RING_S = 2


def _sc_embed(ids3, weight, total_rows):
    nchunk = ids3.shape[1]
    assert nchunk % RING_G == 0 and nchunk >= 2 * RING_G
    mesh = plsc.VectorSubcoreMesh(core_axis_name="c", subcore_axis_name="s")

    @functools.partial(
        pl.kernel,
        mesh=mesh,
        out_type=jax.ShapeDtypeStruct((total_rows, D), jnp.float32),
        scratch_types=[
            pltpu.VMEM((nchunk, CHUNK), jnp.int32),
            pltpu.VMEM((RING_G, CHUNK, D), jnp.float32),
            pltpu.VMEM((RING_S, CHUNK, D), jnp.float32),
            pltpu.SemaphoreType.DMA,
            pltpu.SemaphoreType.DMA,
            pltpu.SemaphoreType.DMA,
            pltpu.SemaphoreType.DMA,
            pltpu.SemaphoreType.DMA,
            pltpu.SemaphoreType.DMA,
        ],
    )
    def k(ids_hbm, w_hbm, out_hbm, idx_v, gbuf, sbuf,
          gsem0, gsem1, gsem2, gsem3, ssem0, ssem1):
        gsems = [gsem0, gsem1, gsem2, gsem3]
        ssems = [ssem0, ssem1]
        wid = lax.axis_index("s") * NC + lax.axis_index("c")
        pltpu.sync_copy(ids_hbm.at[wid], idx_v)
        base = wid * (nchunk * CHUNK)

        def gather(c, bg):
            return pltpu.make_async_copy(
                w_hbm.at[idx_v.at[c]], gbuf.at[bg], gsems[bg])

        def scatter(c, bs):
            return pltpu.make_async_copy(
                sbuf.at[bs], out_hbm.at[pl.ds(base + c * CHUNK, CHUNK)],
                ssems[bs])

        for bg in range(RING_G):
            gather(bg, bg).start()

        def group_body(t, carry):
            g = t * RING_G
            for bg in range(RING_G):
                c = g + bg
                bs = bg % RING_S
                gather(c, bg).wait()

                @pl.when(c >= RING_S)
                def _():
                    scatter(c - RING_S, bs).wait()

                def row_body(i, rcarry):
                    for j in range(D // L):
                        sl = pl.ds(j * L, L)
                        sbuf[bs, i, sl] = gbuf[bg, i, sl] * SCALE
                    return rcarry

                lax.fori_loop(0, CHUNK, row_body, 0)
                scatter(c, bs).start()

                @pl.when(c + RING_G < nchunk)
                def _():
                    gather(c + RING_G, bg).start()
            return carry

        lax.fori_loop(0, nchunk // RING_G, group_body, 0)

        for bs in range(RING_S):
            scatter(nchunk - RING_S + bs, bs).wait()

    return k(ids3, weight)


def kernel(input_ids, weight):
    b, s = input_ids.shape
    total = b * s
    nchunk = total // (NW * CHUNK)
    ids3 = input_ids.astype(jnp.int32).reshape(NW, nchunk, CHUNK)
    out = _sc_embed(ids3, weight, total)
    return out.reshape(b, s, D)


# speedup vs baseline: 1.3848x; 1.3848x over previous
import functools

import jax
import jax.numpy as jnp
from jax import lax
from jax.experimental import pallas as pl
from jax.experimental.pallas import tpu as pltpu
from jax.experimental.pallas import tpu_sc as plsc

D = 1024
L = 16
NC = 2
NS = 16
NW = NC * NS
SCALE = 32.0

CHUNK = 32
RING = 3


def _sc_embed(ids3, weight, total_rows):
    nchunk = ids3.shape[1]
    mesh = plsc.VectorSubcoreMesh(core_axis_name="c", subcore_axis_name="s")

    @functools.partial(
        pl.kernel,
        mesh=mesh,
        out_type=jax.ShapeDtypeStruct((total_rows, D), jnp.float32),
        scratch_types=[
            pltpu.VMEM((nchunk, CHUNK), jnp.int32),
            pltpu.VMEM((RING, CHUNK, D), jnp.float32),
            pltpu.SemaphoreType.DMA,
            pltpu.SemaphoreType.DMA,
            pltpu.SemaphoreType.DMA,
            pltpu.SemaphoreType.DMA,
            pltpu.SemaphoreType.DMA,
            pltpu.SemaphoreType.DMA,
        ],
    )
    def k(ids_hbm, w_hbm, out_hbm, idx_v, buf,
          gsem0, gsem1, gsem2, ssem0, ssem1, ssem2):
        gsems = [gsem0, gsem1, gsem2]
        ssems = [ssem0, ssem1, ssem2]
        wid = lax.axis_index("s") * NC + lax.axis_index("c")
        pltpu.sync_copy(ids_hbm.at[wid], idx_v)
        base = wid * (nchunk * CHUNK)

        def gather(c):
            b = c % RING
            return pltpu.make_async_copy(
                w_hbm.at[idx_v.at[c]], buf.at[b], gsems[b])

        def scatter(c):
            b = c % RING
            return pltpu.make_async_copy(
                buf.at[b], out_hbm.at[pl.ds(base + c * CHUNK, CHUNK)],
                ssems[b])

        gather(0).start()
        gather(1).start()

        for c in range(nchunk):
            b = c % RING
            gather(c).wait()

            def row_body(i, rcarry, _b=b):
                for j in range(D // L):
                    sl = pl.ds(j * L, L)
                    buf[_b, i, sl] = buf[_b, i, sl] * SCALE
                return rcarry

            lax.fori_loop(0, CHUNK, row_body, 0)
            scatter(c).start()
            if c + 2 < nchunk:
                if c - 1 >= 0:
                    scatter(c - 1).wait()
                gather(c + 2).start()

        for c in range(nchunk - RING, nchunk):
            scatter(c).wait()

    return k(ids3, weight)


def kernel(input_ids, weight):
    b, s = input_ids.shape
    total = b * s
    nchunk = total // (NW * CHUNK)
    ids3 = input_ids.astype(jnp.int32).reshape(NW, nchunk, CHUNK)
    out = _sc_embed(ids3, weight, total)
    return out.reshape(b, s, D)
